# initial kernel scaffold (unmeasured)
import jax
import jax.numpy as jnp
from jax import lax
from jax.experimental import pallas as pl
from jax.experimental.pallas import tpu as pltpu

N_DEV = 16


def kernel(x, Win0, Wout0, Win1, Wout1, Win2, Wout2):
    b, d = x.shape
    rows_per = b // N_DEV

    def body(x_ref, win0_ref, wout0_ref, win1_ref, wout1_ref, win2_ref,
             wout2_ref, out_ref, p2_ref, comm0, comm1, comm2,
             s0, r0, s1, r1, s2, r2):
        me = lax.axis_index("i")

        def layer(xin_bf16, win_ref, wout_ref):
            h = jnp.dot(xin_bf16, win_ref[...].astype(jnp.bfloat16),
                        preferred_element_type=jnp.float32)
            h = jnp.maximum(h, 0.0).astype(jnp.bfloat16)
            return jnp.dot(h, wout_ref[...].astype(jnp.bfloat16),
                           preferred_element_type=jnp.float32)

        def all_reduce(partial_f32, comm, s_sems, r_sems):
            comm[0, :, :] = partial_f32.astype(jnp.bfloat16)
            rdmas = []
            for k in range(1, N_DEV):
                rdma = pltpu.make_async_remote_copy(
                    src_ref=comm.at[0],
                    dst_ref=comm.at[k],
                    send_sem=s_sems.at[k],
                    recv_sem=r_sems.at[k],
                    device_id=(lax.rem(me + k, N_DEV),),
                    device_id_type=pl.DeviceIdType.MESH,
                )
                rdma.start()
                rdmas.append(rdma)
            for rdma in rdmas:
                rdma.wait_recv()
            for rdma in rdmas:
                rdma.wait_send()
            return jnp.sum(comm[...].astype(jnp.float32), axis=0)

        x0 = x_ref[...].astype(jnp.bfloat16)
        x1 = all_reduce(layer(x0, win0_ref, wout0_ref), comm0, s0, r0)
        x2 = all_reduce(layer(x1.astype(jnp.bfloat16), win1_ref, wout1_ref),
                        comm1, s1, r1)
        p2_ref[...] = layer(x2.astype(jnp.bfloat16), win2_ref, wout2_ref)

        comm2[0, :, :] = p2_ref[pl.ds(me * rows_per, rows_per), :]
        rdmas = []
        for k in range(1, N_DEV):
            tgt = lax.rem(me + k, N_DEV)
            rdma = pltpu.make_async_remote_copy(
                src_ref=p2_ref.at[pl.ds(tgt * rows_per, rows_per), :],
                dst_ref=comm2.at[k],
                send_sem=s2.at[k],
                recv_sem=r2.at[k],
                device_id=(tgt,),
                device_id_type=pl.DeviceIdType.MESH,
            )
            rdma.start()
            rdmas.append(rdma)
        for rdma in rdmas:
            rdma.wait_recv()
        for rdma in rdmas:
            rdma.wait_send()
        out_ref[...] = jnp.sum(comm2[...], axis=0)

    return pl.pallas_call(
        body,
        out_shape=jax.ShapeDtypeStruct((rows_per, d), jnp.float32),
        in_specs=[pl.BlockSpec(memory_space=pltpu.VMEM)] * 7,
        out_specs=pl.BlockSpec(memory_space=pltpu.VMEM),
        scratch_shapes=[
            pltpu.VMEM((b, d), jnp.float32),
            pltpu.VMEM((N_DEV, b, d), jnp.bfloat16),
            pltpu.VMEM((N_DEV, b, d), jnp.bfloat16),
            pltpu.VMEM((N_DEV, rows_per, d), jnp.float32),
            pltpu.SemaphoreType.DMA((N_DEV,)),
            pltpu.SemaphoreType.DMA((N_DEV,)),
            pltpu.SemaphoreType.DMA((N_DEV,)),
            pltpu.SemaphoreType.DMA((N_DEV,)),
            pltpu.SemaphoreType.DMA((N_DEV,)),
            pltpu.SemaphoreType.DMA((N_DEV,)),
        ],
        compiler_params=pltpu.CompilerParams(collective_id=0),
    )(x, Win0, Wout0, Win1, Wout1, Win2, Wout2)


# baseline (device time: 38124 ns/iter reference)
import jax
import jax.numpy as jnp
from jax import lax
from jax.experimental import pallas as pl
from jax.experimental.pallas import tpu as pltpu

N_DEV = 16


def kernel(x, Win0, Wout0, Win1, Wout1, Win2, Wout2):
    b, d = x.shape
    rows_per = b // N_DEV

    def body(x_ref, win0_ref, wout0_ref, win1_ref, wout1_ref, win2_ref,
             wout2_ref, out_ref, p2_ref, comm0, comm1, comm2,
             s0, r0, s1, r1, s2, r2):
        me = lax.axis_index("i")

        def layer(xin_bf16, win_ref, wout_ref):
            h = jnp.dot(xin_bf16, win_ref[...].astype(jnp.bfloat16),
                        preferred_element_type=jnp.float32)
            h = jnp.maximum(h, 0.0).astype(jnp.bfloat16)
            return jnp.dot(h, wout_ref[...].astype(jnp.bfloat16),
                           preferred_element_type=jnp.float32)

        def all_reduce(partial_f32, comm, s_sems, r_sems):
            comm[0, :, :] = partial_f32.astype(jnp.bfloat16)
            rdmas = []
            for k in range(1, N_DEV):
                rdma = pltpu.make_async_remote_copy(
                    src_ref=comm.at[0],
                    dst_ref=comm.at[k],
                    send_sem=s_sems.at[k],
                    recv_sem=r_sems.at[k],
                    device_id=(lax.rem(me + k, N_DEV),),
                    device_id_type=pl.DeviceIdType.MESH,
                )
                rdma.start()
                rdmas.append(rdma)
            for rdma in rdmas:
                rdma.wait_recv()
            for rdma in rdmas:
                rdma.wait_send()
            return jnp.sum(comm[...].astype(jnp.float32), axis=0)

        x0 = x_ref[...].astype(jnp.bfloat16)
        x1 = all_reduce(layer(x0, win0_ref, wout0_ref), comm0, s0, r0)
        x2 = all_reduce(layer(x1.astype(jnp.bfloat16), win1_ref, wout1_ref),
                        comm1, s1, r1)
        p2_ref[...] = layer(x2.astype(jnp.bfloat16), win2_ref, wout2_ref)

        comm2[0, :, :] = p2_ref[pl.ds(me * rows_per, rows_per), :]
        rdmas = []
        for k in range(1, N_DEV):
            tgt = lax.rem(me + k, N_DEV)
            rdma = pltpu.make_async_remote_copy(
                src_ref=p2_ref.at[pl.ds(tgt * rows_per, rows_per), :],
                dst_ref=comm2.at[k],
                send_sem=s2.at[k],
                recv_sem=r2.at[k],
                device_id=(tgt,),
                device_id_type=pl.DeviceIdType.MESH,
            )
            rdma.start()
            rdmas.append(rdma)
        for rdma in rdmas:
            rdma.wait_recv()
        for rdma in rdmas:
            rdma.wait_send()
        out_ref[...] = jnp.sum(comm2[...], axis=0)

    return pl.pallas_call(
        body,
        out_shape=jax.ShapeDtypeStruct((rows_per, d), jnp.float32),
        in_specs=[pl.BlockSpec(memory_space=pltpu.VMEM)] * 7,
        out_specs=pl.BlockSpec(memory_space=pltpu.VMEM),
        scratch_shapes=[
            pltpu.VMEM((b, d), jnp.float32),
            pltpu.VMEM((N_DEV, b, d), jnp.bfloat16),
            pltpu.VMEM((N_DEV, b, d), jnp.bfloat16),
            pltpu.VMEM((N_DEV, rows_per, d), jnp.float32),
            pltpu.SemaphoreType.DMA((N_DEV,)),
            pltpu.SemaphoreType.DMA((N_DEV,)),
            pltpu.SemaphoreType.DMA((N_DEV,)),
            pltpu.SemaphoreType.DMA((N_DEV,)),
            pltpu.SemaphoreType.DMA((N_DEV,)),
            pltpu.SemaphoreType.DMA((N_DEV,)),
        ],
    )(x, Win0, Wout0, Win1, Wout1, Win2, Wout2)


# device time: 33905 ns/iter; 1.1244x vs baseline; 1.1244x over previous
import jax
import jax.numpy as jnp
from jax import lax
from jax.experimental import pallas as pl
from jax.experimental.pallas import tpu as pltpu

N_DEV = 16


def kernel(x, Win0, Wout0, Win1, Wout1, Win2, Wout2):
    b, d = x.shape
    rows_per = b // N_DEV

    def body(x_ref, win0_ref, wout0_ref, win1_ref, wout1_ref, win2_ref,
             wout2_ref, out_ref, p2_ref, comm0, comm1, comm2,
             s0, r0, s1, r1, s2, r2):
        me = lax.axis_index("i")

        barrier = pltpu.get_barrier_semaphore()
        for k in range(1, N_DEV):
            pl.semaphore_signal(
                barrier, inc=1,
                device_id=(lax.rem(me + k, N_DEV),),
                device_id_type=pl.DeviceIdType.MESH,
            )
        pl.semaphore_wait(barrier, N_DEV - 1)

        def layer(xin_bf16, win_ref, wout_ref):
            h = jnp.dot(xin_bf16, win_ref[...].astype(jnp.bfloat16),
                        preferred_element_type=jnp.float32)
            h = jnp.maximum(h, 0.0).astype(jnp.bfloat16)
            return jnp.dot(h, wout_ref[...].astype(jnp.bfloat16),
                           preferred_element_type=jnp.float32)

        def all_reduce(partial_f32, comm, s_sems, r_sems):
            comm[0, :, :] = partial_f32.astype(jnp.bfloat16)
            rdmas = []
            for k in range(1, N_DEV):
                rdma = pltpu.make_async_remote_copy(
                    src_ref=comm.at[0],
                    dst_ref=comm.at[k],
                    send_sem=s_sems.at[k],
                    recv_sem=r_sems.at[k],
                    device_id=(lax.rem(me + k, N_DEV),),
                    device_id_type=pl.DeviceIdType.MESH,
                )
                rdma.start()
                rdmas.append(rdma)
            for rdma in rdmas:
                rdma.wait_recv()
            for rdma in rdmas:
                rdma.wait_send()
            return jnp.sum(comm[...].astype(jnp.float32), axis=0)

        x0 = x_ref[...].astype(jnp.bfloat16)
        x1 = all_reduce(layer(x0, win0_ref, wout0_ref), comm0, s0, r0)
        x2 = all_reduce(layer(x1.astype(jnp.bfloat16), win1_ref, wout1_ref),
                        comm1, s1, r1)
        p2_ref[...] = layer(x2.astype(jnp.bfloat16), win2_ref, wout2_ref)

        comm2[0, :, :] = p2_ref[pl.ds(me * rows_per, rows_per), :]
        rdmas = []
        for k in range(1, N_DEV):
            tgt = lax.rem(me + k, N_DEV)
            rdma = pltpu.make_async_remote_copy(
                src_ref=p2_ref.at[pl.ds(tgt * rows_per, rows_per), :],
                dst_ref=comm2.at[k],
                send_sem=s2.at[k],
                recv_sem=r2.at[k],
                device_id=(tgt,),
                device_id_type=pl.DeviceIdType.MESH,
            )
            rdma.start()
            rdmas.append(rdma)
        for rdma in rdmas:
            rdma.wait_recv()
        for rdma in rdmas:
            rdma.wait_send()
        out_ref[...] = jnp.sum(comm2[...], axis=0)

    return pl.pallas_call(
        body,
        out_shape=jax.ShapeDtypeStruct((rows_per, d), jnp.float32),
        in_specs=[pl.BlockSpec(memory_space=pltpu.VMEM)] * 7,
        out_specs=pl.BlockSpec(memory_space=pltpu.VMEM),
        scratch_shapes=[
            pltpu.VMEM((b, d), jnp.float32),
            pltpu.VMEM((N_DEV, b, d), jnp.bfloat16),
            pltpu.VMEM((N_DEV, b, d), jnp.bfloat16),
            pltpu.VMEM((N_DEV, rows_per, d), jnp.float32),
            pltpu.SemaphoreType.DMA((N_DEV,)),
            pltpu.SemaphoreType.DMA((N_DEV,)),
            pltpu.SemaphoreType.DMA((N_DEV,)),
            pltpu.SemaphoreType.DMA((N_DEV,)),
            pltpu.SemaphoreType.DMA((N_DEV,)),
            pltpu.SemaphoreType.DMA((N_DEV,)),
        ],
        compiler_params=pltpu.CompilerParams(collective_id=0),
    )(x, Win0, Wout0, Win1, Wout1, Win2, Wout2)


# device time: 29162 ns/iter; 1.3073x vs baseline; 1.1626x over previous
import jax
import jax.numpy as jnp
from jax import lax
from jax.experimental import pallas as pl
from jax.experimental.pallas import tpu as pltpu

N_DEV = 16


def kernel(x, Win0, Wout0, Win1, Wout1, Win2, Wout2):
    b, d = x.shape
    rows_per = b // N_DEV

    def body(x_ref, win0_ref, wout0_ref, win1_ref, wout1_ref, win2_ref,
             wout2_ref, out_ref, p2_ref,
             commA0, commB0, commA1, commB1, comm2,
             sa0, ra0, sb0, rb0, sa1, ra1, sb1, rb1, s2, r2):
        me = lax.axis_index("i")
        z = lax.div(me, 4)
        q = lax.rem(me, 4)

        barrier = pltpu.get_barrier_semaphore()
        for k in range(1, N_DEV):
            pl.semaphore_signal(
                barrier, inc=1,
                device_id=(lax.rem(me + k, N_DEV),),
                device_id_type=pl.DeviceIdType.MESH,
            )

        def layer(xin_bf16, win_ref, wout_ref):
            h = jnp.dot(xin_bf16, win_ref[...].astype(jnp.bfloat16),
                        preferred_element_type=jnp.float32)
            h = jnp.maximum(h, 0.0).astype(jnp.bfloat16)
            return jnp.dot(h, wout_ref[...].astype(jnp.bfloat16),
                           preferred_element_type=jnp.float32)

        def group_oneshot(partial_f32, comm, s_sems, r_sems, peer_of):
            comm[0, :, :] = partial_f32.astype(jnp.bfloat16)
            rdmas = []
            for k in range(1, 4):
                rdma = pltpu.make_async_remote_copy(
                    src_ref=comm.at[0],
                    dst_ref=comm.at[k],
                    send_sem=s_sems.at[k],
                    recv_sem=r_sems.at[k],
                    device_id=(peer_of(k),),
                    device_id_type=pl.DeviceIdType.MESH,
                )
                rdma.start()
                rdmas.append(rdma)
            for rdma in rdmas:
                rdma.wait_recv()
            for rdma in rdmas:
                rdma.wait_send()
            return jnp.sum(comm[...].astype(jnp.float32), axis=0)

        def all_reduce(partial_f32, commA, commB, sa, ra, sb, rb):
            plane_sum = group_oneshot(
                partial_f32, commA, sa, ra,
                lambda k: z * 4 + lax.rem(q + k, 4))
            return group_oneshot(
                plane_sum, commB, sb, rb,
                lambda k: lax.rem(z + k, 4) * 4 + q)

        x0 = x_ref[...].astype(jnp.bfloat16)
        p0 = layer(x0, win0_ref, wout0_ref)
        pl.semaphore_wait(barrier, N_DEV - 1)

        x1 = all_reduce(p0, commA0, commB0, sa0, ra0, sb0, rb0)
        x2 = all_reduce(layer(x1.astype(jnp.bfloat16), win1_ref, wout1_ref),
                        commA1, commB1, sa1, ra1, sb1, rb1)
        p2_ref[...] = layer(x2.astype(jnp.bfloat16), win2_ref, wout2_ref)

        comm2[0, :, :] = p2_ref[pl.ds(me * rows_per, rows_per), :]
        rdmas = []
        for k in range(1, N_DEV):
            tgt = lax.rem(me + k, N_DEV)
            rdma = pltpu.make_async_remote_copy(
                src_ref=p2_ref.at[pl.ds(tgt * rows_per, rows_per), :],
                dst_ref=comm2.at[k],
                send_sem=s2.at[k],
                recv_sem=r2.at[k],
                device_id=(tgt,),
                device_id_type=pl.DeviceIdType.MESH,
            )
            rdma.start()
            rdmas.append(rdma)
        for rdma in rdmas:
            rdma.wait_recv()
        for rdma in rdmas:
            rdma.wait_send()
        out_ref[...] = jnp.sum(comm2[...], axis=0)

    return pl.pallas_call(
        body,
        out_shape=jax.ShapeDtypeStruct((rows_per, d), jnp.float32),
        in_specs=[pl.BlockSpec(memory_space=pltpu.VMEM)] * 7,
        out_specs=pl.BlockSpec(memory_space=pltpu.VMEM),
        scratch_shapes=[
            pltpu.VMEM((b, d), jnp.float32),
            pltpu.VMEM((4, b, d), jnp.bfloat16),
            pltpu.VMEM((4, b, d), jnp.bfloat16),
            pltpu.VMEM((4, b, d), jnp.bfloat16),
            pltpu.VMEM((4, b, d), jnp.bfloat16),
            pltpu.VMEM((N_DEV, rows_per, d), jnp.float32),
            pltpu.SemaphoreType.DMA((4,)),
            pltpu.SemaphoreType.DMA((4,)),
            pltpu.SemaphoreType.DMA((4,)),
            pltpu.SemaphoreType.DMA((4,)),
            pltpu.SemaphoreType.DMA((4,)),
            pltpu.SemaphoreType.DMA((4,)),
            pltpu.SemaphoreType.DMA((4,)),
            pltpu.SemaphoreType.DMA((4,)),
            pltpu.SemaphoreType.DMA((N_DEV,)),
            pltpu.SemaphoreType.DMA((N_DEV,)),
        ],
        compiler_params=pltpu.CompilerParams(collective_id=0),
    )(x, Win0, Wout0, Win1, Wout1, Win2, Wout2)
